# Initial kernel scaffold; baseline (speedup 1.0000x reference)
import jax
import jax.numpy as jnp
from jax.experimental import pallas as pl


def kernel(x, edge_index, batch, params):
    raise NotImplementedError("write your pallas kernel here")


# speedup vs baseline: 3.5686x; 3.5686x over previous
import jax
import jax.numpy as jnp
from jax import lax
from jax.experimental import pallas as pl
from jax.experimental.pallas import tpu as pltpu
from jax.experimental.pallas import tpu_sc as plsc

_N = 10000
_E = 320000
_D = 128
_G = 128
_NB = 3
_NM = 3

_NC = 2
_NS = 16
_NW = _NC * _NS
_EPT = _E // _NW
_CH = 80
_NCH = _EPT // _CH
_HN = _N // 2
_AR = _HN + 8
_RPS = 320
_RPL = _HN - (_NS - 1) * _RPS
_ZR = 40


def _fill_rows(ref, nrows, value):
  def _st(i, _):
    ref[i // 8, pl.ds((i % 8) * 16, 16)] = jnp.full((16,), value, jnp.float32)
    return 0
  lax.fori_loop(0, nrows * 8, _st, 0)


def _zero_acc(acc_sh, zrow_v, s):
  def _z(k, _):
    pltpu.sync_copy(zrow_v, acc_sh.at[pl.ds(s * _RPS + k * _ZR, _ZR)])
    return 0

  @pl.when(s < _NS - 1)
  def _full():
    lax.fori_loop(0, _RPS // _ZR, _z, 0)

  @pl.when(s == _NS - 1)
  def _last():
    lax.fori_loop(0, _RPL // _ZR, _z, 0)
    pltpu.sync_copy(zrow_v.at[pl.ds(0, 8)], acc_sh.at[pl.ds(_HN, 8)])


def _remap_dst(dst_v, adj_v, base):
  def _rm(i, _):
    j = i // (_CH // 16)
    k = (i % (_CH // 16)) * 16
    d = dst_v[j, pl.ds(k, 16)]
    rel = d - base
    ok = jnp.logical_and(rel >= 0, rel < _HN)
    adj_v[j, pl.ds(k, 16)] = jnp.where(ok, rel, _HN + (d & 7))
    return 0
  lax.fori_loop(0, _NCH * (_CH // 16), _rm, 0)


def _readout(acc_sh, out_hbm, c, s, base):
  @pl.when(s < _NS - 1)
  def _full():
    pltpu.sync_copy(acc_sh.at[pl.ds(s * _RPS, _RPS)],
                    out_hbm.at[c, pl.ds(base + s * _RPS, _RPS)])

  @pl.when(s == _NS - 1)
  def _last():
    pltpu.sync_copy(acc_sh.at[pl.ds(s * _RPS, _RPL)],
                    out_hbm.at[c, pl.ds(base + s * _RPS, _RPL)])


def _make_segsum():
  mesh = plsc.VectorSubcoreMesh(core_axis_name="c", subcore_axis_name="s")
  scratch = [
      pltpu.VMEM((_NCH, _CH), jnp.int32),
      pltpu.VMEM((_NCH, _CH), jnp.int32),
      pltpu.VMEM((_NCH, _CH), jnp.int32),
      pltpu.VMEM((_CH, _D), jnp.float32),
      pltpu.VMEM((_ZR, _D), jnp.float32),
      pltpu.VMEM_SHARED((_AR, _D), jnp.float32),
      pltpu.SemaphoreType.DMA,
  ]

  def body(x_hbm, src_hbm, dst_hbm, acc_out,
           src_v, dst_v, adj_v, rows_v, zrow_v, acc_sh, sem):
    c = lax.axis_index("c")
    s = lax.axis_index("s")
    wid = s * _NC + c

    _fill_rows(zrow_v, _ZR, 0.0)
    _zero_acc(acc_sh, zrow_v, s)

    pltpu.sync_copy(src_hbm.at[wid], src_v)
    pltpu.sync_copy(dst_hbm.at[wid], dst_v)

    for half in range(2):
      _remap_dst(dst_v, adj_v, half * _HN)
      plsc.subcore_barrier()

      def _edge(j, _):
        pltpu.async_copy(x_hbm.at[src_v.at[j]], rows_v, sem).wait()
        pltpu.sync_copy(rows_v, acc_sh.at[adj_v.at[j]], add=True)
        return 0
      lax.fori_loop(0, _NCH, _edge, 0)

      plsc.subcore_barrier()
      _readout(acc_sh, acc_out, c, s, half * _HN)
      if half == 0:
        _zero_acc(acc_sh, zrow_v, s)

  return pl.kernel(
      body,
      out_type=jax.ShapeDtypeStruct((_NC, _N, _D), jnp.float32),
      mesh=mesh,
      scratch_types=scratch,
  )


def _make_deg():
  mesh = plsc.VectorSubcoreMesh(core_axis_name="c", subcore_axis_name="s")
  scratch = [
      pltpu.VMEM((_NCH, _CH), jnp.int32),
      pltpu.VMEM((_NCH, _CH), jnp.int32),
      pltpu.VMEM((_CH, _D), jnp.float32),
      pltpu.VMEM((_ZR, _D), jnp.float32),
      pltpu.VMEM_SHARED((_AR, _D), jnp.float32),
  ]

  def body(dst_hbm, deg_out, dst_v, adj_v, ones_v, zrow_v, acc_sh):
    c = lax.axis_index("c")
    s = lax.axis_index("s")
    wid = s * _NC + c

    _fill_rows(zrow_v, _ZR, 0.0)
    _fill_rows(ones_v, _CH, 1.0)
    _zero_acc(acc_sh, zrow_v, s)

    pltpu.sync_copy(dst_hbm.at[wid], dst_v)

    for half in range(2):
      _remap_dst(dst_v, adj_v, half * _HN)
      plsc.subcore_barrier()

      def _edge(j, _):
        pltpu.sync_copy(ones_v, acc_sh.at[adj_v.at[j]], add=True)
        return 0
      lax.fori_loop(0, _NCH, _edge, 0)

      plsc.subcore_barrier()
      _readout(acc_sh, deg_out, c, s, half * _HN)
      if half == 0:
        _zero_acc(acc_sh, zrow_v, s)

  return pl.kernel(
      body,
      out_type=jax.ShapeDtypeStruct((_NC, _N, _D), jnp.float32),
      mesh=mesh,
      scratch_types=scratch,
  )


_SEGSUM = _make_segsum()
_DEG = _make_deg()

_R = 2000


def _ln(t, g, b):
  m = jnp.mean(t, axis=-1, keepdims=True)
  xm = t - m
  v = jnp.mean(xm * xm, axis=-1, keepdims=True)
  return xm / jnp.sqrt(v + 1e-5) * g + b


def _elu(v):
  return jnp.where(v > 0, v, jnp.exp(v) - 1.0)


def _dense_body(x_ref, pa_ref, pb_ref, da_ref, db_ref,
                wl_ref, bl_ref, wr_ref, br_ref, g1_ref, be1_ref,
                w1_ref, b1_ref, w2_ref, b2_ref, g2_ref, be2_ref, o_ref):
  x = x_ref[...]
  sagg = pa_ref[0] + pb_ref[0]
  deg = jnp.maximum(da_ref[0][:, :1] + db_ref[0][:, :1], 1.0)
  mean = sagg / deg
  y = (jnp.dot(mean, wl_ref[...], preferred_element_type=jnp.float32)
       + bl_ref[...]
       + jnp.dot(x, wr_ref[...], preferred_element_type=jnp.float32)
       + br_ref[...])
  y1 = _ln(y + x, g1_ref[...], be1_ref[...])
  h = _elu(jnp.dot(y1, w1_ref[...], preferred_element_type=jnp.float32)
           + b1_ref[...])
  h = jnp.dot(h, w2_ref[...], preferred_element_type=jnp.float32) + b2_ref[...]
  o_ref[...] = _ln(h + y1, g2_ref[...], be2_ref[...])


def _dense_block(x, acc, degp, weights):
  row = lambda i: (i, 0)
  fixed = lambda i: (0, 0)
  in_specs = [
      pl.BlockSpec((_R, _D), row),
      pl.BlockSpec((1, _R, _D), lambda i: (0, i, 0)),
      pl.BlockSpec((1, _R, _D), lambda i: (1, i, 0)),
      pl.BlockSpec((1, _R, _D), lambda i: (0, i, 0)),
      pl.BlockSpec((1, _R, _D), lambda i: (1, i, 0)),
  ]
  for a in weights:
    in_specs.append(pl.BlockSpec(a.shape, fixed))
  return pl.pallas_call(
      _dense_body,
      grid=(_N // _R,),
      in_specs=in_specs,
      out_specs=pl.BlockSpec((_R, _D), row),
      out_shape=jax.ShapeDtypeStruct((_N, _D), jnp.float32),
  )(x, acc, acc, degp, degp, *weights)


def _pool_body(x_ref, b_ref, w1_ref, c1_ref, w2_ref, c2_ref, w3_ref, c3_ref,
               wh_ref, ch_ref, o_ref, acc_ref, cnt_ref):
  i = pl.program_id(0)

  @pl.when(i == 0)
  def _init():
    acc_ref[...] = jnp.zeros_like(acc_ref)
    cnt_ref[...] = jnp.zeros_like(cnt_ref)

  b = b_ref[...]
  gids = lax.broadcasted_iota(jnp.int32, (1, _G), 1).astype(jnp.float32)
  onehot = (b == gids).astype(jnp.float32)
  dn = (((0,), (0,)), ((), ()))
  acc_ref[...] += lax.dot_general(onehot, x_ref[...], dn,
                                  preferred_element_type=jnp.float32)
  cnt_ref[...] += lax.dot_general(onehot, jnp.ones((_R, _D), jnp.float32), dn,
                                  preferred_element_type=jnp.float32)

  @pl.when(i == pl.num_programs(0) - 1)
  def _fin():
    pooled = acc_ref[...] / jnp.maximum(cnt_ref[...], 1.0)
    for w_r, c_r in ((w1_ref, c1_ref), (w2_ref, c2_ref), (w3_ref, c3_ref)):
      pooled = _elu(jnp.dot(pooled, w_r[...],
                            preferred_element_type=jnp.float32) + c_r[...])
    o_ref[...] = (jnp.dot(pooled, wh_ref[...],
                          preferred_element_type=jnp.float32) + ch_ref[...])


def _pool_mlp(x, bf, weights):
  fixed = lambda i: (0, 0)
  in_specs = [pl.BlockSpec((_R, _D), lambda i: (i, 0)),
              pl.BlockSpec((_R, 1), lambda i: (i, 0))]
  for a in weights:
    in_specs.append(pl.BlockSpec(a.shape, fixed))
  return pl.pallas_call(
      _pool_body,
      grid=(_N // _R,),
      in_specs=in_specs,
      out_specs=pl.BlockSpec((_G, _D), fixed),
      out_shape=jax.ShapeDtypeStruct((_G, _D), jnp.float32),
      scratch_shapes=[
          pltpu.VMEM((_G, _D), jnp.float32),
          pltpu.VMEM((_G, _D), jnp.float32),
      ],
  )(x, bf, *weights)


def kernel(x, edge_index, batch, params):
  src = edge_index[0].reshape(_NW, _NCH, _CH)
  dst = edge_index[1].reshape(_NW, _NCH, _CH)
  bf = batch.astype(jnp.float32).reshape(_N, 1)
  it = iter(params)

  degp = _DEG(dst)
  xcur = x
  for _ in range(_NB):
    (wl, bl, wr, br, g1, be1, w1, b1, w2, b2, g2, be2) = (
        next(it) for _ in range(12))
    acc = _SEGSUM(xcur, src, dst)
    weights = (wl, bl.reshape(1, _D), wr, br.reshape(1, _D),
               g1.reshape(1, _D), be1.reshape(1, _D),
               w1, b1.reshape(1, _D), w2, b2.reshape(1, _D),
               g2.reshape(1, _D), be2.reshape(1, _D))
    xcur = _dense_block(xcur, acc, degp, weights)

  mlp = []
  for _ in range(_NM):
    w = next(it)
    b = next(it)
    mlp += [w, b.reshape(1, _D)]
  wh = next(it)
  bh = next(it)
  whp = jnp.pad(wh, ((0, 0), (0, _D - 2)))
  bhp = jnp.pad(bh, (0, _D - 2)).reshape(1, _D)

  raw = _pool_mlp(xcur, bf, (*mlp, whp, bhp))[:, :2]
  out = raw.reshape(-1, 1, 2)
  return out[..., 0], out[..., 1]


# speedup vs baseline: 5.7038x; 1.5983x over previous
import jax
import jax.numpy as jnp
from jax import lax
from jax.experimental import pallas as pl
from jax.experimental.pallas import tpu as pltpu
from jax.experimental.pallas import tpu_sc as plsc

_N = 10000
_E = 320000
_D = 128
_G = 128
_NB = 3
_NM = 3

_NC = 2
_NS = 16
_NW = _NC * _NS
_EPT = _E // _NW
_CH = 80
_NCH = _EPT // _CH
_HN = _N // 2
_AR = _HN + 8
_RPS = 320
_RPL = _HN - (_NS - 1) * _RPS
_ZR = 40


def _fill_rows(ref, nrows, value):
  def _st(i, _):
    ref[i // 8, pl.ds((i % 8) * 16, 16)] = jnp.full((16,), value, jnp.float32)
    return 0
  lax.fori_loop(0, nrows * 8, _st, 0)


def _zero_acc(acc_sh, zrow_v, s):
  def _z(k, _):
    pltpu.sync_copy(zrow_v, acc_sh.at[pl.ds(s * _RPS + k * _ZR, _ZR)])
    return 0

  @pl.when(s < _NS - 1)
  def _full():
    lax.fori_loop(0, _RPS // _ZR, _z, 0)

  @pl.when(s == _NS - 1)
  def _last():
    lax.fori_loop(0, _RPL // _ZR, _z, 0)
    pltpu.sync_copy(zrow_v.at[pl.ds(0, 8)], acc_sh.at[pl.ds(_HN, 8)])


def _filter_edges(dst_v, fdst_v, base, src_v=None, fsrc_v=None):
  lane = lax.iota(jnp.int32, 16)

  def _f(i, cnt):
    j = i // (_CH // 16)
    k = (i % (_CH // 16)) * 16
    d = dst_v[j, pl.ds(k, 16)]
    rel = d - base
    ok = jnp.logical_and(rel >= 0, rel < _HN)
    pref = plsc.cumsum(ok.astype(jnp.int32))
    pos = jnp.where(ok, cnt + pref - 1, _EPT + _CH + lane)
    plsc.store_scatter(fdst_v, [pos], rel)
    if fsrc_v is not None:
      sv = src_v[j, pl.ds(k, 16)]
      plsc.store_scatter(fsrc_v, [pos], sv)
    return cnt + jnp.max(pref)

  cnt = lax.fori_loop(0, _EPT // 16, _f, 0)
  spread = lax.iota(jnp.int32, 16) & 7
  for k in range(_CH // 16):
    fdst_v[pl.ds(cnt + k * 16, 16)] = spread + _HN
    if fsrc_v is not None:
      fsrc_v[pl.ds(cnt + k * 16, 16)] = spread
  return (cnt + _CH - 1) // _CH


def _readout(acc_sh, out_hbm, c, s, base):
  @pl.when(s < _NS - 1)
  def _full():
    pltpu.sync_copy(acc_sh.at[pl.ds(s * _RPS, _RPS)],
                    out_hbm.at[c, pl.ds(base + s * _RPS, _RPS)])

  @pl.when(s == _NS - 1)
  def _last():
    pltpu.sync_copy(acc_sh.at[pl.ds(s * _RPS, _RPL)],
                    out_hbm.at[c, pl.ds(base + s * _RPS, _RPL)])


def _make_segsum():
  mesh = plsc.VectorSubcoreMesh(core_axis_name="c", subcore_axis_name="s")
  scratch = [
      pltpu.VMEM((_NCH, _CH), jnp.int32),
      pltpu.VMEM((_NCH, _CH), jnp.int32),
      pltpu.VMEM((_EPT + _CH + 16,), jnp.int32),
      pltpu.VMEM((_EPT + _CH + 16,), jnp.int32),
      pltpu.VMEM((_CH, _D), jnp.float32),
      pltpu.VMEM((_ZR, _D), jnp.float32),
      pltpu.VMEM_SHARED((_AR, _D), jnp.float32),
      pltpu.SemaphoreType.DMA,
  ]

  def body(x_hbm, src_hbm, dst_hbm, acc_out,
           src_v, dst_v, fsrc_v, fdst_v, rows_v, zrow_v, acc_sh, sem):
    c = lax.axis_index("c")
    s = lax.axis_index("s")
    wid = s * _NC + c

    _fill_rows(zrow_v, _ZR, 0.0)
    _zero_acc(acc_sh, zrow_v, s)

    pltpu.sync_copy(src_hbm.at[wid], src_v)
    pltpu.sync_copy(dst_hbm.at[wid], dst_v)

    for half in range(2):
      nch = _filter_edges(dst_v, fdst_v, half * _HN, src_v, fsrc_v)
      plsc.subcore_barrier()

      def _edge(j, _):
        pltpu.async_copy(x_hbm.at[fsrc_v.at[pl.ds(j * _CH, _CH)]],
                         rows_v, sem).wait()
        pltpu.sync_copy(rows_v, acc_sh.at[fdst_v.at[pl.ds(j * _CH, _CH)]],
                        add=True)
        return 0
      lax.fori_loop(0, nch, _edge, 0)

      plsc.subcore_barrier()
      _readout(acc_sh, acc_out, c, s, half * _HN)
      if half == 0:
        _zero_acc(acc_sh, zrow_v, s)

  return pl.kernel(
      body,
      out_type=jax.ShapeDtypeStruct((_NC, _N, _D), jnp.float32),
      mesh=mesh,
      scratch_types=scratch,
      compiler_params=pltpu.CompilerParams(needs_layout_passes=False),
  )


def _make_deg():
  mesh = plsc.VectorSubcoreMesh(core_axis_name="c", subcore_axis_name="s")
  scratch = [
      pltpu.VMEM((_NCH, _CH), jnp.int32),
      pltpu.VMEM((_EPT + _CH + 16,), jnp.int32),
      pltpu.VMEM((_CH, _D), jnp.float32),
      pltpu.VMEM((_ZR, _D), jnp.float32),
      pltpu.VMEM_SHARED((_AR, _D), jnp.float32),
  ]

  def body(dst_hbm, deg_out, dst_v, fdst_v, ones_v, zrow_v, acc_sh):
    c = lax.axis_index("c")
    s = lax.axis_index("s")
    wid = s * _NC + c

    _fill_rows(zrow_v, _ZR, 0.0)
    _fill_rows(ones_v, _CH, 1.0)
    _zero_acc(acc_sh, zrow_v, s)

    pltpu.sync_copy(dst_hbm.at[wid], dst_v)

    for half in range(2):
      nch = _filter_edges(dst_v, fdst_v, half * _HN)
      plsc.subcore_barrier()

      def _edge(j, _):
        pltpu.sync_copy(ones_v, acc_sh.at[fdst_v.at[pl.ds(j * _CH, _CH)]],
                        add=True)
        return 0
      lax.fori_loop(0, nch, _edge, 0)

      plsc.subcore_barrier()
      _readout(acc_sh, deg_out, c, s, half * _HN)
      if half == 0:
        _zero_acc(acc_sh, zrow_v, s)

  return pl.kernel(
      body,
      out_type=jax.ShapeDtypeStruct((_NC, _N, _D), jnp.float32),
      mesh=mesh,
      scratch_types=scratch,
      compiler_params=pltpu.CompilerParams(needs_layout_passes=False),
  )


_SEGSUM = _make_segsum()
_DEG = _make_deg()

_R = 2000


def _ln(t, g, b):
  m = jnp.mean(t, axis=-1, keepdims=True)
  xm = t - m
  v = jnp.mean(xm * xm, axis=-1, keepdims=True)
  return xm / jnp.sqrt(v + 1e-5) * g + b


def _elu(v):
  return jnp.where(v > 0, v, jnp.exp(v) - 1.0)


def _dense_body(x_ref, pa_ref, pb_ref, da_ref, db_ref,
                wl_ref, bl_ref, wr_ref, br_ref, g1_ref, be1_ref,
                w1_ref, b1_ref, w2_ref, b2_ref, g2_ref, be2_ref, o_ref):
  x = x_ref[...]
  sagg = pa_ref[0] + pb_ref[0]
  deg = jnp.maximum(da_ref[0][:, :1] + db_ref[0][:, :1], 1.0)
  mean = sagg / deg
  y = (jnp.dot(mean, wl_ref[...], preferred_element_type=jnp.float32)
       + bl_ref[...]
       + jnp.dot(x, wr_ref[...], preferred_element_type=jnp.float32)
       + br_ref[...])
  y1 = _ln(y + x, g1_ref[...], be1_ref[...])
  h = _elu(jnp.dot(y1, w1_ref[...], preferred_element_type=jnp.float32)
           + b1_ref[...])
  h = jnp.dot(h, w2_ref[...], preferred_element_type=jnp.float32) + b2_ref[...]
  o_ref[...] = _ln(h + y1, g2_ref[...], be2_ref[...])


def _dense_block(x, acc, degp, weights):
  row = lambda i: (i, 0)
  fixed = lambda i: (0, 0)
  in_specs = [
      pl.BlockSpec((_R, _D), row),
      pl.BlockSpec((1, _R, _D), lambda i: (0, i, 0)),
      pl.BlockSpec((1, _R, _D), lambda i: (1, i, 0)),
      pl.BlockSpec((1, _R, _D), lambda i: (0, i, 0)),
      pl.BlockSpec((1, _R, _D), lambda i: (1, i, 0)),
  ]
  for a in weights:
    in_specs.append(pl.BlockSpec(a.shape, fixed))
  return pl.pallas_call(
      _dense_body,
      grid=(_N // _R,),
      in_specs=in_specs,
      out_specs=pl.BlockSpec((_R, _D), row),
      out_shape=jax.ShapeDtypeStruct((_N, _D), jnp.float32),
  )(x, acc, acc, degp, degp, *weights)


def _pool_body(x_ref, b_ref, w1_ref, c1_ref, w2_ref, c2_ref, w3_ref, c3_ref,
               wh_ref, ch_ref, o_ref, acc_ref, cnt_ref):
  i = pl.program_id(0)

  @pl.when(i == 0)
  def _init():
    acc_ref[...] = jnp.zeros_like(acc_ref)
    cnt_ref[...] = jnp.zeros_like(cnt_ref)

  b = b_ref[...]
  gids = lax.broadcasted_iota(jnp.int32, (1, _G), 1).astype(jnp.float32)
  onehot = (b == gids).astype(jnp.float32)
  dn = (((0,), (0,)), ((), ()))
  acc_ref[...] += lax.dot_general(onehot, x_ref[...], dn,
                                  preferred_element_type=jnp.float32)
  cnt_ref[...] += lax.dot_general(onehot, jnp.ones((_R, _D), jnp.float32), dn,
                                  preferred_element_type=jnp.float32)

  @pl.when(i == pl.num_programs(0) - 1)
  def _fin():
    pooled = acc_ref[...] / jnp.maximum(cnt_ref[...], 1.0)
    for w_r, c_r in ((w1_ref, c1_ref), (w2_ref, c2_ref), (w3_ref, c3_ref)):
      pooled = _elu(jnp.dot(pooled, w_r[...],
                            preferred_element_type=jnp.float32) + c_r[...])
    o_ref[...] = (jnp.dot(pooled, wh_ref[...],
                          preferred_element_type=jnp.float32) + ch_ref[...])


def _pool_mlp(x, bf, weights):
  fixed = lambda i: (0, 0)
  in_specs = [pl.BlockSpec((_R, _D), lambda i: (i, 0)),
              pl.BlockSpec((_R, 1), lambda i: (i, 0))]
  for a in weights:
    in_specs.append(pl.BlockSpec(a.shape, fixed))
  return pl.pallas_call(
      _pool_body,
      grid=(_N // _R,),
      in_specs=in_specs,
      out_specs=pl.BlockSpec((_G, _D), fixed),
      out_shape=jax.ShapeDtypeStruct((_G, _D), jnp.float32),
      scratch_shapes=[
          pltpu.VMEM((_G, _D), jnp.float32),
          pltpu.VMEM((_G, _D), jnp.float32),
      ],
  )(x, bf, *weights)


def kernel(x, edge_index, batch, params):
  src = edge_index[0].reshape(_NW, _NCH, _CH)
  dst = edge_index[1].reshape(_NW, _NCH, _CH)
  bf = batch.astype(jnp.float32).reshape(_N, 1)
  it = iter(params)

  degp = _DEG(dst)
  xcur = x
  for _ in range(_NB):
    (wl, bl, wr, br, g1, be1, w1, b1, w2, b2, g2, be2) = (
        next(it) for _ in range(12))
    acc = _SEGSUM(xcur, src, dst)
    weights = (wl, bl.reshape(1, _D), wr, br.reshape(1, _D),
               g1.reshape(1, _D), be1.reshape(1, _D),
               w1, b1.reshape(1, _D), w2, b2.reshape(1, _D),
               g2.reshape(1, _D), be2.reshape(1, _D))
    xcur = _dense_block(xcur, acc, degp, weights)

  mlp = []
  for _ in range(_NM):
    w = next(it)
    b = next(it)
    mlp += [w, b.reshape(1, _D)]
  wh = next(it)
  bh = next(it)
  whp = jnp.pad(wh, ((0, 0), (0, _D - 2)))
  bhp = jnp.pad(bh, (0, _D - 2)).reshape(1, _D)

  raw = _pool_mlp(xcur, bf, (*mlp, whp, bhp))[:, :2]
  out = raw.reshape(-1, 1, 2)
  return out[..., 0], out[..., 1]


# speedup vs baseline: 6.2296x; 1.0922x over previous
import jax
import jax.numpy as jnp
from jax import lax
from jax.experimental import pallas as pl
from jax.experimental.pallas import tpu as pltpu
from jax.experimental.pallas import tpu_sc as plsc

_N = 10000
_E = 320000
_D = 128
_G = 128
_NB = 3
_NM = 3

_NC = 2
_NS = 16
_NW = _NC * _NS
_EPT = _E // _NW
_CH = 80
_NCH = _EPT // _CH
_PS = 2560
_NPASS = 4
_NP = _PS * _NPASS
_AR = _PS + 8
_RPS = _PS // _NS
_ZR = 40
_K = 2
_GE = _K * _CH


def _fill_rows(ref, nrows, value):
  def _st(i, _):
    ref[i // 8, pl.ds((i % 8) * 16, 16)] = jnp.full((16,), value, jnp.float32)
    return 0
  lax.fori_loop(0, nrows * 8, _st, 0)


def _zero_acc(acc_sh, zrow_v, s):
  def _z(k, _):
    pltpu.sync_copy(zrow_v, acc_sh.at[pl.ds(s * _RPS + k * _ZR, _ZR)])
    return 0
  lax.fori_loop(0, _RPS // _ZR, _z, 0)

  @pl.when(s == _NS - 1)
  def _last():
    pltpu.sync_copy(zrow_v.at[pl.ds(0, 8)], acc_sh.at[pl.ds(_PS, 8)])


def _filter_edges(dst_v, fdst_v, base, src_v=None, fsrc_v=None):
  lane = lax.iota(jnp.int32, 16)

  def _f(i, cnt):
    j = i // (_CH // 16)
    k = (i % (_CH // 16)) * 16
    d = dst_v[j, pl.ds(k, 16)]
    rel = d - base
    ok = jnp.logical_and(rel >= 0, rel < _PS)
    pref = plsc.cumsum(ok.astype(jnp.int32))
    pos = jnp.where(ok, cnt + pref - 1, _EPT + _GE + lane)
    plsc.store_scatter(fdst_v, [pos], rel)
    if fsrc_v is not None:
      sv = src_v[j, pl.ds(k, 16)]
      plsc.store_scatter(fsrc_v, [pos], sv)
    return cnt + jnp.max(pref)

  cnt = lax.fori_loop(0, _EPT // 16, _f, 0)
  spread = lax.iota(jnp.int32, 16) & 7
  for k in range(_GE // 16):
    fdst_v[pl.ds(cnt + k * 16, 16)] = spread + _PS
    if fsrc_v is not None:
      fsrc_v[pl.ds(cnt + k * 16, 16)] = spread
  return jnp.maximum((cnt + _GE - 1) // _GE, 1)


def _readout(acc_sh, out_hbm, c, s, base):
  pltpu.sync_copy(acc_sh.at[pl.ds(s * _RPS, _RPS)],
                  out_hbm.at[c, pl.ds(base + s * _RPS, _RPS)])


def _make_segsum(with_deg):
  mesh = plsc.VectorSubcoreMesh(core_axis_name="c", subcore_axis_name="s")
  out_type = [jax.ShapeDtypeStruct((_NC, _NP, _D), jnp.float32)]
  scratch = [
      pltpu.VMEM((_NCH, _CH), jnp.int32),
      pltpu.VMEM((_NCH, _CH), jnp.int32),
      pltpu.VMEM((_EPT + _GE + 16,), jnp.int32),
      pltpu.VMEM((_EPT + _GE + 16,), jnp.int32),
      pltpu.VMEM((_K, _CH, _D), jnp.float32),
      pltpu.VMEM((_K, _CH, _D), jnp.float32),
      pltpu.VMEM((_ZR, _D), jnp.float32),
      pltpu.VMEM_SHARED((_AR, _D), jnp.float32),
      pltpu.SemaphoreType.DMA,
      pltpu.SemaphoreType.DMA,
  ]
  if with_deg:
    out_type.append(jax.ShapeDtypeStruct((_NC, _NP, _D), jnp.float32))
    scratch.append(pltpu.VMEM((_CH, _D), jnp.float32))

  def body(x_hbm, src_hbm, dst_hbm, *rest):
    if with_deg:
      (acc_out, deg_out, src_v, dst_v, fsrc_v, fdst_v, buf_a, buf_b,
       zrow_v, acc_sh, gsem, ssem, ones_v) = rest
    else:
      (acc_out, src_v, dst_v, fsrc_v, fdst_v, buf_a, buf_b,
       zrow_v, acc_sh, gsem, ssem) = rest
    c = lax.axis_index("c")
    s = lax.axis_index("s")
    wid = s * _NC + c

    _fill_rows(zrow_v, _ZR, 0.0)
    if with_deg:
      _fill_rows(ones_v, _CH, 1.0)

    pltpu.sync_copy(src_hbm.at[wid], src_v)
    pltpu.sync_copy(dst_hbm.at[wid], dst_v)

    def _fire_gathers(g, buf):
      for i in range(_K):
        pltpu.async_copy(
            x_hbm.at[fsrc_v.at[pl.ds(g * _GE + i * _CH, _CH)]],
            buf.at[i], gsem)

    def _drain_gathers(g, buf):
      for i in range(_K):
        pltpu.make_async_copy(
            x_hbm.at[fsrc_v.at[pl.ds(g * _GE + i * _CH, _CH)]],
            buf.at[i], gsem).wait()

    def _pass(p, _):
      base = p * _PS
      _zero_acc(acc_sh, zrow_v, s)
      ngr = _filter_edges(dst_v, fdst_v, base, src_v, fsrc_v)
      plsc.subcore_barrier()

      _fire_gathers(0, buf_a)

      def _group(g, _):
        def _phase(buf_x, buf_y):
          _drain_gathers(g, buf_x)

          @pl.when(g + 1 < ngr)
          def _next():
            _fire_gathers(g + 1, buf_y)

          for i in range(_K):
            pltpu.sync_copy(
                buf_x.at[i],
                acc_sh.at[fdst_v.at[pl.ds(g * _GE + i * _CH, _CH)]],
                add=True)

        @pl.when(g % 2 == 0)
        def _even():
          _phase(buf_a, buf_b)

        @pl.when(g % 2 == 1)
        def _odd():
          _phase(buf_b, buf_a)
        return 0
      lax.fori_loop(0, ngr, _group, 0)

      plsc.subcore_barrier()
      _readout(acc_sh, acc_out, c, s, base)

      if with_deg:
        _zero_acc(acc_sh, zrow_v, s)
        plsc.subcore_barrier()

        def _dgroup(g, _):
          hs = [
              pltpu.async_copy(
                  ones_v,
                  acc_sh.at[fdst_v.at[pl.ds(g * _GE + i * _CH, _CH)]],
                  ssem, add=True)
              for i in range(_K)
          ]
          for h in hs:
            h.wait()
          return 0
        lax.fori_loop(0, ngr, _dgroup, 0)

        plsc.subcore_barrier()
        _readout(acc_sh, deg_out, c, s, base)
      return 0

    lax.fori_loop(0, _NPASS, _pass, 0)

  if not with_deg:
    out_type = out_type[0]
  return pl.kernel(
      body,
      out_type=out_type,
      mesh=mesh,
      scratch_types=scratch,
      compiler_params=pltpu.CompilerParams(needs_layout_passes=False),
  )


_SEGSUM_DEG = _make_segsum(True)
_SEGSUM = _make_segsum(False)

_R = 2000


def _ln(t, g, b):
  m = jnp.mean(t, axis=-1, keepdims=True)
  xm = t - m
  v = jnp.mean(xm * xm, axis=-1, keepdims=True)
  return xm / jnp.sqrt(v + 1e-5) * g + b


def _elu(v):
  return jnp.where(v > 0, v, jnp.exp(v) - 1.0)


def _dense_body(x_ref, pa_ref, pb_ref, da_ref, db_ref,
                wl_ref, bl_ref, wr_ref, br_ref, g1_ref, be1_ref,
                w1_ref, b1_ref, w2_ref, b2_ref, g2_ref, be2_ref, o_ref):
  x = x_ref[...]
  sagg = pa_ref[0] + pb_ref[0]
  deg = jnp.maximum(da_ref[0][:, :1] + db_ref[0][:, :1], 1.0)
  mean = sagg / deg
  y = (jnp.dot(mean, wl_ref[...], preferred_element_type=jnp.float32)
       + bl_ref[...]
       + jnp.dot(x, wr_ref[...], preferred_element_type=jnp.float32)
       + br_ref[...])
  y1 = _ln(y + x, g1_ref[...], be1_ref[...])
  h = _elu(jnp.dot(y1, w1_ref[...], preferred_element_type=jnp.float32)
           + b1_ref[...])
  h = jnp.dot(h, w2_ref[...], preferred_element_type=jnp.float32) + b2_ref[...]
  o_ref[...] = _ln(h + y1, g2_ref[...], be2_ref[...])


def _dense_block(x, acc, degp, weights):
  row = lambda i: (i, 0)
  fixed = lambda i: (0, 0)
  in_specs = [
      pl.BlockSpec((_R, _D), row),
      pl.BlockSpec((1, _R, _D), lambda i: (0, i, 0)),
      pl.BlockSpec((1, _R, _D), lambda i: (1, i, 0)),
      pl.BlockSpec((1, _R, _D), lambda i: (0, i, 0)),
      pl.BlockSpec((1, _R, _D), lambda i: (1, i, 0)),
  ]
  for a in weights:
    in_specs.append(pl.BlockSpec(a.shape, fixed))
  return pl.pallas_call(
      _dense_body,
      grid=(_N // _R,),
      in_specs=in_specs,
      out_specs=pl.BlockSpec((_R, _D), row),
      out_shape=jax.ShapeDtypeStruct((_N, _D), jnp.float32),
  )(x, acc, acc, degp, degp, *weights)


def _pool_body(x_ref, b_ref, w1_ref, c1_ref, w2_ref, c2_ref, w3_ref, c3_ref,
               wh_ref, ch_ref, o_ref, acc_ref, cnt_ref):
  i = pl.program_id(0)

  @pl.when(i == 0)
  def _init():
    acc_ref[...] = jnp.zeros_like(acc_ref)
    cnt_ref[...] = jnp.zeros_like(cnt_ref)

  b = b_ref[...]
  gids = lax.broadcasted_iota(jnp.int32, (1, _G), 1).astype(jnp.float32)
  onehot = (b == gids).astype(jnp.float32)
  dn = (((0,), (0,)), ((), ()))
  acc_ref[...] += lax.dot_general(onehot, x_ref[...], dn,
                                  preferred_element_type=jnp.float32)
  cnt_ref[...] += lax.dot_general(onehot, jnp.ones((_R, _D), jnp.float32), dn,
                                  preferred_element_type=jnp.float32)

  @pl.when(i == pl.num_programs(0) - 1)
  def _fin():
    pooled = acc_ref[...] / jnp.maximum(cnt_ref[...], 1.0)
    for w_r, c_r in ((w1_ref, c1_ref), (w2_ref, c2_ref), (w3_ref, c3_ref)):
      pooled = _elu(jnp.dot(pooled, w_r[...],
                            preferred_element_type=jnp.float32) + c_r[...])
    o_ref[...] = (jnp.dot(pooled, wh_ref[...],
                          preferred_element_type=jnp.float32) + ch_ref[...])


def _pool_mlp(x, bf, weights):
  fixed = lambda i: (0, 0)
  in_specs = [pl.BlockSpec((_R, _D), lambda i: (i, 0)),
              pl.BlockSpec((_R, 1), lambda i: (i, 0))]
  for a in weights:
    in_specs.append(pl.BlockSpec(a.shape, fixed))
  return pl.pallas_call(
      _pool_body,
      grid=(_N // _R,),
      in_specs=in_specs,
      out_specs=pl.BlockSpec((_G, _D), fixed),
      out_shape=jax.ShapeDtypeStruct((_G, _D), jnp.float32),
      scratch_shapes=[
          pltpu.VMEM((_G, _D), jnp.float32),
          pltpu.VMEM((_G, _D), jnp.float32),
      ],
  )(x, bf, *weights)


def kernel(x, edge_index, batch, params):
  src = edge_index[0].reshape(_NW, _NCH, _CH)
  dst = edge_index[1].reshape(_NW, _NCH, _CH)
  bf = batch.astype(jnp.float32).reshape(_N, 1)
  it = iter(params)

  xcur = x
  degp = None
  for _ in range(_NB):
    (wl, bl, wr, br, g1, be1, w1, b1, w2, b2, g2, be2) = (
        next(it) for _ in range(12))
    if degp is None:
      acc, degp = _SEGSUM_DEG(xcur, src, dst)
    else:
      acc = _SEGSUM(xcur, src, dst)
    weights = (wl, bl.reshape(1, _D), wr, br.reshape(1, _D),
               g1.reshape(1, _D), be1.reshape(1, _D),
               w1, b1.reshape(1, _D), w2, b2.reshape(1, _D),
               g2.reshape(1, _D), be2.reshape(1, _D))
    xcur = _dense_block(xcur, acc, degp, weights)

  mlp = []
  for _ in range(_NM):
    w = next(it)
    b = next(it)
    mlp += [w, b.reshape(1, _D)]
  wh = next(it)
  bh = next(it)
  whp = jnp.pad(wh, ((0, 0), (0, _D - 2)))
  bhp = jnp.pad(bh, (0, _D - 2)).reshape(1, _D)

  raw = _pool_mlp(xcur, bf, (*mlp, whp, bhp))[:, :2]
  out = raw.reshape(-1, 1, 2)
  return out[..., 0], out[..., 1]


# speedup vs baseline: 6.2518x; 1.0036x over previous
import jax
import jax.numpy as jnp
from jax import lax
from jax.experimental import pallas as pl
from jax.experimental.pallas import tpu as pltpu
from jax.experimental.pallas import tpu_sc as plsc

_N = 10000
_E = 320000
_D = 128
_G = 128
_NB = 3
_NM = 3

_NC = 2
_NS = 16
_NW = _NC * _NS
_EPT = _E // _NW
_CH = 80
_NCH = _EPT // _CH
_PS = 2560
_NPASS = 4
_NP = _PS * _NPASS
_AR = _PS + 8
_RPS = _PS // _NS
_ZR = 40
_K = 2
_GE = _K * _CH


def _fill_rows(ref, nrows, value):
  def _st(i, _):
    ref[i // 8, pl.ds((i % 8) * 16, 16)] = jnp.full((16,), value, jnp.float32)
    return 0
  lax.fori_loop(0, nrows * 8, _st, 0)


def _zero_acc(acc_sh, zrow_v, s):
  def _z(k, _):
    pltpu.sync_copy(zrow_v, acc_sh.at[pl.ds(s * _RPS + k * _ZR, _ZR)])
    return 0
  lax.fori_loop(0, _RPS // _ZR, _z, 0)

  @pl.when(s == _NS - 1)
  def _last():
    pltpu.sync_copy(zrow_v.at[pl.ds(0, 8)], acc_sh.at[pl.ds(_PS, 8)])


def _filter_edges(dst_v, fdst_v, base, src_v=None, fsrc_v=None):
  lane = lax.iota(jnp.int32, 16)

  def _f(i, cnt):
    j = i // (_CH // 16)
    k = (i % (_CH // 16)) * 16
    d = dst_v[j, pl.ds(k, 16)]
    rel = d - base
    ok = jnp.logical_and(rel >= 0, rel < _PS)
    pref = plsc.cumsum(ok.astype(jnp.int32))
    pos = jnp.where(ok, cnt + pref - 1, _EPT + _GE + lane)
    plsc.store_scatter(fdst_v, [pos], rel)
    if fsrc_v is not None:
      sv = src_v[j, pl.ds(k, 16)]
      plsc.store_scatter(fsrc_v, [pos], sv)
    return cnt + jnp.max(pref)

  cnt = lax.fori_loop(0, _EPT // 16, _f, 0)
  spread = lax.iota(jnp.int32, 16) & 7
  for k in range(_GE // 16):
    fdst_v[pl.ds(cnt + k * 16, 16)] = spread + _PS
    if fsrc_v is not None:
      fsrc_v[pl.ds(cnt + k * 16, 16)] = spread
  return jnp.maximum((cnt + _GE - 1) // _GE, 1)


def _readout(acc_sh, out_hbm, c, s, base):
  pltpu.sync_copy(acc_sh.at[pl.ds(s * _RPS, _RPS)],
                  out_hbm.at[c, pl.ds(base + s * _RPS, _RPS)])


def _make_segsum(with_deg):
  mesh = plsc.VectorSubcoreMesh(core_axis_name="c", subcore_axis_name="s")
  out_type = [jax.ShapeDtypeStruct((_NC, _NP, _D), jnp.float32)]
  scratch = [
      pltpu.VMEM((_NCH, _CH), jnp.int32),
      pltpu.VMEM((_NCH, _CH), jnp.int32),
      pltpu.VMEM((_EPT + _GE + 16,), jnp.int32),
      pltpu.VMEM((_EPT + _GE + 16,), jnp.int32),
      pltpu.VMEM((_K, _CH, _D), jnp.float32),
      pltpu.VMEM((_K, _CH, _D), jnp.float32),
      pltpu.VMEM((_ZR, _D), jnp.float32),
      pltpu.VMEM_SHARED((_AR, _D), jnp.float32),
      pltpu.SemaphoreType.DMA,
      pltpu.SemaphoreType.DMA,
  ]
  if with_deg:
    out_type.append(jax.ShapeDtypeStruct((_NC, _NP, _D), jnp.float32))
    scratch.append(pltpu.VMEM((_CH, _D), jnp.float32))

  def body(x_hbm, src_hbm, dst_hbm, *rest):
    if with_deg:
      (acc_out, deg_out, src_v, dst_v, fsrc_v, fdst_v, buf_a, buf_b,
       zrow_v, acc_sh, gsem, ssem, ones_v) = rest
    else:
      (acc_out, src_v, dst_v, fsrc_v, fdst_v, buf_a, buf_b,
       zrow_v, acc_sh, gsem, ssem) = rest
    c = lax.axis_index("c")
    s = lax.axis_index("s")
    wid = s * _NC + c

    _fill_rows(zrow_v, _ZR, 0.0)
    if with_deg:
      _fill_rows(ones_v, _CH, 1.0)

    pltpu.sync_copy(src_hbm.at[wid], src_v)
    pltpu.sync_copy(dst_hbm.at[wid], dst_v)

    def _fire_gathers(g, buf):
      for i in range(_K):
        pltpu.async_copy(
            x_hbm.at[fsrc_v.at[pl.ds(g * _GE + i * _CH, _CH)]],
            buf.at[i], gsem)

    def _drain_gathers(g, buf):
      for i in range(_K):
        pltpu.make_async_copy(
            x_hbm.at[fsrc_v.at[pl.ds(g * _GE + i * _CH, _CH)]],
            buf.at[i], gsem).wait()

    def _pass(p, _):
      base = p * _PS
      _zero_acc(acc_sh, zrow_v, s)
      ngr = _filter_edges(dst_v, fdst_v, base, src_v, fsrc_v)
      plsc.subcore_barrier()

      _fire_gathers(0, buf_a)

      def _group(g, _):
        def _phase(buf_x, buf_y):
          _drain_gathers(g, buf_x)
          hs = [
              pltpu.async_copy(
                  buf_x.at[i],
                  acc_sh.at[fdst_v.at[pl.ds(g * _GE + i * _CH, _CH)]],
                  ssem, add=True)
              for i in range(_K)
          ]

          @pl.when(g + 1 < ngr)
          def _next():
            _fire_gathers(g + 1, buf_y)

          for h in hs:
            h.wait()

        @pl.when(g % 2 == 0)
        def _even():
          _phase(buf_a, buf_b)

        @pl.when(g % 2 == 1)
        def _odd():
          _phase(buf_b, buf_a)
        return 0
      lax.fori_loop(0, ngr, _group, 0)

      plsc.subcore_barrier()
      _readout(acc_sh, acc_out, c, s, base)

      if with_deg:
        _zero_acc(acc_sh, zrow_v, s)
        plsc.subcore_barrier()

        def _dgroup(g, _):
          hs = [
              pltpu.async_copy(
                  ones_v,
                  acc_sh.at[fdst_v.at[pl.ds(g * _GE + i * _CH, _CH)]],
                  ssem, add=True)
              for i in range(_K)
          ]
          for h in hs:
            h.wait()
          return 0
        lax.fori_loop(0, ngr, _dgroup, 0)

        plsc.subcore_barrier()
        _readout(acc_sh, deg_out, c, s, base)
      return 0

    lax.fori_loop(0, _NPASS, _pass, 0)

  if not with_deg:
    out_type = out_type[0]
  return pl.kernel(
      body,
      out_type=out_type,
      mesh=mesh,
      scratch_types=scratch,
      compiler_params=pltpu.CompilerParams(needs_layout_passes=False),
  )


_SEGSUM_DEG = _make_segsum(True)
_SEGSUM = _make_segsum(False)

_R = 2000


def _ln(t, g, b):
  m = jnp.mean(t, axis=-1, keepdims=True)
  xm = t - m
  v = jnp.mean(xm * xm, axis=-1, keepdims=True)
  return xm / jnp.sqrt(v + 1e-5) * g + b


def _elu(v):
  return jnp.where(v > 0, v, jnp.exp(v) - 1.0)


def _dense_body(x_ref, pa_ref, pb_ref, da_ref, db_ref,
                wl_ref, bl_ref, wr_ref, br_ref, g1_ref, be1_ref,
                w1_ref, b1_ref, w2_ref, b2_ref, g2_ref, be2_ref, o_ref):
  x = x_ref[...]
  sagg = pa_ref[0] + pb_ref[0]
  deg = jnp.maximum(da_ref[0][:, :1] + db_ref[0][:, :1], 1.0)
  mean = sagg / deg
  y = (jnp.dot(mean, wl_ref[...], preferred_element_type=jnp.float32)
       + bl_ref[...]
       + jnp.dot(x, wr_ref[...], preferred_element_type=jnp.float32)
       + br_ref[...])
  y1 = _ln(y + x, g1_ref[...], be1_ref[...])
  h = _elu(jnp.dot(y1, w1_ref[...], preferred_element_type=jnp.float32)
           + b1_ref[...])
  h = jnp.dot(h, w2_ref[...], preferred_element_type=jnp.float32) + b2_ref[...]
  o_ref[...] = _ln(h + y1, g2_ref[...], be2_ref[...])


def _dense_block(x, acc, degp, weights):
  row = lambda i: (i, 0)
  fixed = lambda i: (0, 0)
  in_specs = [
      pl.BlockSpec((_R, _D), row),
      pl.BlockSpec((1, _R, _D), lambda i: (0, i, 0)),
      pl.BlockSpec((1, _R, _D), lambda i: (1, i, 0)),
      pl.BlockSpec((1, _R, _D), lambda i: (0, i, 0)),
      pl.BlockSpec((1, _R, _D), lambda i: (1, i, 0)),
  ]
  for a in weights:
    in_specs.append(pl.BlockSpec(a.shape, fixed))
  return pl.pallas_call(
      _dense_body,
      grid=(_N // _R,),
      in_specs=in_specs,
      out_specs=pl.BlockSpec((_R, _D), row),
      out_shape=jax.ShapeDtypeStruct((_N, _D), jnp.float32),
  )(x, acc, acc, degp, degp, *weights)


def _pool_body(x_ref, b_ref, w1_ref, c1_ref, w2_ref, c2_ref, w3_ref, c3_ref,
               wh_ref, ch_ref, o_ref, acc_ref, cnt_ref):
  i = pl.program_id(0)

  @pl.when(i == 0)
  def _init():
    acc_ref[...] = jnp.zeros_like(acc_ref)
    cnt_ref[...] = jnp.zeros_like(cnt_ref)

  b = b_ref[...]
  gids = lax.broadcasted_iota(jnp.int32, (1, _G), 1).astype(jnp.float32)
  onehot = (b == gids).astype(jnp.float32)
  dn = (((0,), (0,)), ((), ()))
  acc_ref[...] += lax.dot_general(onehot, x_ref[...], dn,
                                  preferred_element_type=jnp.float32)
  cnt_ref[...] += lax.dot_general(onehot, jnp.ones((_R, _D), jnp.float32), dn,
                                  preferred_element_type=jnp.float32)

  @pl.when(i == pl.num_programs(0) - 1)
  def _fin():
    pooled = acc_ref[...] / jnp.maximum(cnt_ref[...], 1.0)
    for w_r, c_r in ((w1_ref, c1_ref), (w2_ref, c2_ref), (w3_ref, c3_ref)):
      pooled = _elu(jnp.dot(pooled, w_r[...],
                            preferred_element_type=jnp.float32) + c_r[...])
    o_ref[...] = (jnp.dot(pooled, wh_ref[...],
                          preferred_element_type=jnp.float32) + ch_ref[...])


def _pool_mlp(x, bf, weights):
  fixed = lambda i: (0, 0)
  in_specs = [pl.BlockSpec((_R, _D), lambda i: (i, 0)),
              pl.BlockSpec((_R, 1), lambda i: (i, 0))]
  for a in weights:
    in_specs.append(pl.BlockSpec(a.shape, fixed))
  return pl.pallas_call(
      _pool_body,
      grid=(_N // _R,),
      in_specs=in_specs,
      out_specs=pl.BlockSpec((_G, _D), fixed),
      out_shape=jax.ShapeDtypeStruct((_G, _D), jnp.float32),
      scratch_shapes=[
          pltpu.VMEM((_G, _D), jnp.float32),
          pltpu.VMEM((_G, _D), jnp.float32),
      ],
  )(x, bf, *weights)


def kernel(x, edge_index, batch, params):
  src = edge_index[0].reshape(_NW, _NCH, _CH)
  dst = edge_index[1].reshape(_NW, _NCH, _CH)
  bf = batch.astype(jnp.float32).reshape(_N, 1)
  it = iter(params)

  xcur = x
  degp = None
  for _ in range(_NB):
    (wl, bl, wr, br, g1, be1, w1, b1, w2, b2, g2, be2) = (
        next(it) for _ in range(12))
    if degp is None:
      acc, degp = _SEGSUM_DEG(xcur, src, dst)
    else:
      acc = _SEGSUM(xcur, src, dst)
    weights = (wl, bl.reshape(1, _D), wr, br.reshape(1, _D),
               g1.reshape(1, _D), be1.reshape(1, _D),
               w1, b1.reshape(1, _D), w2, b2.reshape(1, _D),
               g2.reshape(1, _D), be2.reshape(1, _D))
    xcur = _dense_block(xcur, acc, degp, weights)

  mlp = []
  for _ in range(_NM):
    w = next(it)
    b = next(it)
    mlp += [w, b.reshape(1, _D)]
  wh = next(it)
  bh = next(it)
  whp = jnp.pad(wh, ((0, 0), (0, _D - 2)))
  bhp = jnp.pad(bh, (0, _D - 2)).reshape(1, _D)

  raw = _pool_mlp(xcur, bf, (*mlp, whp, bhp))[:, :2]
  out = raw.reshape(-1, 1, 2)
  return out[..., 0], out[..., 1]


# speedup vs baseline: 6.2666x; 1.0024x over previous
import jax
import jax.numpy as jnp
from jax import lax
from jax.experimental import pallas as pl
from jax.experimental.pallas import tpu as pltpu
from jax.experimental.pallas import tpu_sc as plsc

_N = 10000
_E = 320000
_D = 128
_G = 128
_NB = 3
_NM = 3

_NC = 2
_NS = 16
_NW = _NC * _NS
_EPT = _E // _NW
_CH = 80
_NCH = _EPT // _CH
_PS = 2560
_NPASS = 4
_NP = _PS * _NPASS
_AR = _PS + 8
_RPS = _PS // _NS
_ZR = 40
_K = 2
_GE = _K * _CH
_TRASH = _EPT + _GE
_FL = _TRASH + 16


def _fill_rows(ref, nrows, value):
  def _st(i, _):
    ref[i // 8, pl.ds((i % 8) * 16, 16)] = jnp.full((16,), value, jnp.float32)
    return 0
  lax.fori_loop(0, nrows * 8, _st, 0)


def _zero_acc(acc_sh, zrow_v, s):
  def _z(k, _):
    pltpu.sync_copy(zrow_v, acc_sh.at[pl.ds(s * _RPS + k * _ZR, _ZR)])
    return 0
  lax.fori_loop(0, _RPS // _ZR, _z, 0)

  @pl.when(s == _NS - 1)
  def _last():
    pltpu.sync_copy(zrow_v.at[pl.ds(0, 8)], acc_sh.at[pl.ds(_PS, 8)])


def _filter_edges(dst_v, fdst_v, base, src_v, fsrc_v):
  lane = lax.iota(jnp.int32, 16)

  def _f(i, cnt):
    j = i // (_CH // 16)
    k = (i % (_CH // 16)) * 16
    d = dst_v[j, pl.ds(k, 16)]
    rel = d - base
    ok = jnp.logical_and(rel >= 0, rel < _PS)
    pref = plsc.cumsum(ok.astype(jnp.int32))
    pos = jnp.where(ok, cnt + pref - 1, _TRASH + lane)
    plsc.store_scatter(fdst_v, [pos], rel)
    sv = src_v[j, pl.ds(k, 16)]
    plsc.store_scatter(fsrc_v, [pos], sv)
    return cnt + jnp.max(pref)

  cnt = lax.fori_loop(0, _EPT // 16, _f, 0)
  spread = lax.iota(jnp.int32, 16) & 7
  for k in range(_GE // 16):
    fdst_v[pl.ds(cnt + k * 16, 16)] = spread + _PS
    fsrc_v[pl.ds(cnt + k * 16, 16)] = spread
  return jnp.maximum((cnt + _GE - 1) // _GE, 1)


def _readout(acc_sh, out_hbm, c, s, base):
  pltpu.sync_copy(acc_sh.at[pl.ds(s * _RPS, _RPS)],
                  out_hbm.at[c, pl.ds(base + s * _RPS, _RPS)])


def _make_segsum(build):
  mesh = plsc.VectorSubcoreMesh(core_axis_name="c", subcore_axis_name="s")
  out_type = [jax.ShapeDtypeStruct((_NC, _NP, _D), jnp.float32)]
  scratch = [
      pltpu.VMEM((_FL,), jnp.int32),
      pltpu.VMEM((_FL,), jnp.int32),
      pltpu.VMEM((_K, _CH, _D), jnp.float32),
      pltpu.VMEM((_K, _CH, _D), jnp.float32),
      pltpu.VMEM((_ZR, _D), jnp.float32),
      pltpu.VMEM_SHARED((_AR, _D), jnp.float32),
      pltpu.SemaphoreType.DMA,
      pltpu.SemaphoreType.DMA,
  ]
  scratch += [
      pltpu.VMEM((_NCH, _CH), jnp.int32),
      pltpu.VMEM((_NCH, _CH), jnp.int32),
  ]
  if build:
    out_type.append(jax.ShapeDtypeStruct((_NC, _NP, _D), jnp.float32))
    scratch.append(pltpu.VMEM((_CH, _D), jnp.float32))

  def body(x_hbm, src_hbm, dst_hbm, *rest):
    if build:
      (acc_out, deg_out,
       fsrc_v, fdst_v, buf_a, buf_b, zrow_v, acc_sh, gsem, ssem,
       src_v, dst_v, ones_v) = rest
    else:
      (acc_out,
       fsrc_v, fdst_v, buf_a, buf_b, zrow_v, acc_sh, gsem, ssem,
       src_v, dst_v) = rest
    c = lax.axis_index("c")
    s = lax.axis_index("s")
    wid = s * _NC + c

    _fill_rows(zrow_v, _ZR, 0.0)
    if build:
      _fill_rows(ones_v, _CH, 1.0)
    pltpu.sync_copy(src_hbm.at[wid], src_v)
    pltpu.sync_copy(dst_hbm.at[wid], dst_v)

    def _fire_gathers(j0, buf):
      for i in range(_K):
        pltpu.async_copy(
            x_hbm.at[fsrc_v.at[pl.ds(j0 * _GE + i * _CH, _CH)]],
            buf.at[i], gsem)

    def _drain_gathers(j0, buf):
      for i in range(_K):
        pltpu.make_async_copy(
            x_hbm.at[fsrc_v.at[pl.ds(j0 * _GE + i * _CH, _CH)]],
            buf.at[i], gsem).wait()

    def _pass(p, _):
      base = p * _PS
      _zero_acc(acc_sh, zrow_v, s)
      ngr = _filter_edges(dst_v, fdst_v, base, src_v, fsrc_v)
      plsc.subcore_barrier()

      _fire_gathers(0, buf_a)

      def _group(g, _):
        def _phase(buf_x, buf_y):
          _drain_gathers(g, buf_x)
          hs = [
              pltpu.async_copy(
                  buf_x.at[i],
                  acc_sh.at[fdst_v.at[pl.ds(g * _GE + i * _CH, _CH)]],
                  ssem, add=True)
              for i in range(_K)
          ]

          @pl.when(g + 1 < ngr)
          def _next():
            _fire_gathers(g + 1, buf_y)

          for h in hs:
            h.wait()

        @pl.when(g % 2 == 0)
        def _even():
          _phase(buf_a, buf_b)

        @pl.when(g % 2 == 1)
        def _odd():
          _phase(buf_b, buf_a)
        return 0
      lax.fori_loop(0, ngr, _group, 0)

      plsc.subcore_barrier()
      _readout(acc_sh, acc_out, c, s, base)

      if build:
        _zero_acc(acc_sh, zrow_v, s)
        plsc.subcore_barrier()

        def _dgroup(g, _):
          hs = [
              pltpu.async_copy(
                  ones_v,
                  acc_sh.at[fdst_v.at[pl.ds(g * _GE + i * _CH, _CH)]],
                  ssem, add=True)
              for i in range(_K)
          ]
          for h in hs:
            h.wait()
          return 0
        lax.fori_loop(0, ngr, _dgroup, 0)

        plsc.subcore_barrier()
        _readout(acc_sh, deg_out, c, s, base)
      return 0

    lax.fori_loop(0, _NPASS, _pass, 0)

  if not build:
    out_type = out_type[0]
  return pl.kernel(
      body,
      out_type=out_type,
      mesh=mesh,
      scratch_types=scratch,
      compiler_params=pltpu.CompilerParams(needs_layout_passes=False),
  )


_SEGSUM_DEG = _make_segsum(True)
_SEGSUM = _make_segsum(False)

_R = 2000


def _ln(t, g, b):
  m = jnp.mean(t, axis=-1, keepdims=True)
  xm = t - m
  v = jnp.mean(xm * xm, axis=-1, keepdims=True)
  return xm / jnp.sqrt(v + 1e-5) * g + b


def _elu(v):
  return jnp.where(v > 0, v, jnp.exp(v) - 1.0)


def _dense_body(x_ref, pa_ref, pb_ref, da_ref, db_ref,
                wl_ref, bl_ref, wr_ref, br_ref, g1_ref, be1_ref,
                w1_ref, b1_ref, w2_ref, b2_ref, g2_ref, be2_ref, o_ref):
  x = x_ref[...]
  sagg = pa_ref[0] + pb_ref[0]
  deg = jnp.maximum(da_ref[0][:, :1] + db_ref[0][:, :1], 1.0)
  mean = sagg / deg
  y = (jnp.dot(mean, wl_ref[...], preferred_element_type=jnp.float32)
       + bl_ref[...]
       + jnp.dot(x, wr_ref[...], preferred_element_type=jnp.float32)
       + br_ref[...])
  y1 = _ln(y + x, g1_ref[...], be1_ref[...])
  h = _elu(jnp.dot(y1, w1_ref[...], preferred_element_type=jnp.float32)
           + b1_ref[...])
  h = jnp.dot(h, w2_ref[...], preferred_element_type=jnp.float32) + b2_ref[...]
  o_ref[...] = _ln(h + y1, g2_ref[...], be2_ref[...])


def _dense_block(x, acc, degp, weights):
  row = lambda i: (i, 0)
  fixed = lambda i: (0, 0)
  in_specs = [
      pl.BlockSpec((_R, _D), row),
      pl.BlockSpec((1, _R, _D), lambda i: (0, i, 0)),
      pl.BlockSpec((1, _R, _D), lambda i: (1, i, 0)),
      pl.BlockSpec((1, _R, _D), lambda i: (0, i, 0)),
      pl.BlockSpec((1, _R, _D), lambda i: (1, i, 0)),
  ]
  for a in weights:
    in_specs.append(pl.BlockSpec(a.shape, fixed))
  return pl.pallas_call(
      _dense_body,
      grid=(_N // _R,),
      in_specs=in_specs,
      out_specs=pl.BlockSpec((_R, _D), row),
      out_shape=jax.ShapeDtypeStruct((_N, _D), jnp.float32),
  )(x, acc, acc, degp, degp, *weights)


def _pool_body(x_ref, b_ref, w1_ref, c1_ref, w2_ref, c2_ref, w3_ref, c3_ref,
               wh_ref, ch_ref, o_ref, acc_ref, cnt_ref):
  i = pl.program_id(0)

  @pl.when(i == 0)
  def _init():
    acc_ref[...] = jnp.zeros_like(acc_ref)
    cnt_ref[...] = jnp.zeros_like(cnt_ref)

  b = b_ref[...]
  gids = lax.broadcasted_iota(jnp.int32, (1, _G), 1).astype(jnp.float32)
  onehot = (b == gids).astype(jnp.float32)
  dn = (((0,), (0,)), ((), ()))
  acc_ref[...] += lax.dot_general(onehot, x_ref[...], dn,
                                  preferred_element_type=jnp.float32)
  cnt_ref[...] += lax.dot_general(onehot, jnp.ones((_R, _D), jnp.float32), dn,
                                  preferred_element_type=jnp.float32)

  @pl.when(i == pl.num_programs(0) - 1)
  def _fin():
    pooled = acc_ref[...] / jnp.maximum(cnt_ref[...], 1.0)
    for w_r, c_r in ((w1_ref, c1_ref), (w2_ref, c2_ref), (w3_ref, c3_ref)):
      pooled = _elu(jnp.dot(pooled, w_r[...],
                            preferred_element_type=jnp.float32) + c_r[...])
    o_ref[...] = (jnp.dot(pooled, wh_ref[...],
                          preferred_element_type=jnp.float32) + ch_ref[...])


def _pool_mlp(x, bf, weights):
  fixed = lambda i: (0, 0)
  in_specs = [pl.BlockSpec((_R, _D), lambda i: (i, 0)),
              pl.BlockSpec((_R, 1), lambda i: (i, 0))]
  for a in weights:
    in_specs.append(pl.BlockSpec(a.shape, fixed))
  return pl.pallas_call(
      _pool_body,
      grid=(_N // _R,),
      in_specs=in_specs,
      out_specs=pl.BlockSpec((_G, _D), fixed),
      out_shape=jax.ShapeDtypeStruct((_G, _D), jnp.float32),
      scratch_shapes=[
          pltpu.VMEM((_G, _D), jnp.float32),
          pltpu.VMEM((_G, _D), jnp.float32),
      ],
  )(x, bf, *weights)


def kernel(x, edge_index, batch, params):
  src = edge_index[0].reshape(_NW, _NCH, _CH)
  dst = edge_index[1].reshape(_NW, _NCH, _CH)
  bf = batch.astype(jnp.float32).reshape(_N, 1)
  it = iter(params)

  xcur = x
  degp = None
  for _ in range(_NB):
    (wl, bl, wr, br, g1, be1, w1, b1, w2, b2, g2, be2) = (
        next(it) for _ in range(12))
    if degp is None:
      acc, degp = _SEGSUM_DEG(xcur, src, dst)
    else:
      acc = _SEGSUM(xcur, src, dst)
    weights = (wl, bl.reshape(1, _D), wr, br.reshape(1, _D),
               g1.reshape(1, _D), be1.reshape(1, _D),
               w1, b1.reshape(1, _D), w2, b2.reshape(1, _D),
               g2.reshape(1, _D), be2.reshape(1, _D))
    xcur = _dense_block(xcur, acc, degp, weights)

  mlp = []
  for _ in range(_NM):
    w = next(it)
    b = next(it)
    mlp += [w, b.reshape(1, _D)]
  wh = next(it)
  bh = next(it)
  whp = jnp.pad(wh, ((0, 0), (0, _D - 2)))
  bhp = jnp.pad(bh, (0, _D - 2)).reshape(1, _D)

  raw = _pool_mlp(xcur, bf, (*mlp, whp, bhp))[:, :2]
  out = raw.reshape(-1, 1, 2)
  return out[..., 0], out[..., 1]


# speedup vs baseline: 6.2713x; 1.0008x over previous
import jax
import jax.numpy as jnp
from jax import lax
from jax.experimental import pallas as pl
from jax.experimental.pallas import tpu as pltpu
from jax.experimental.pallas import tpu_sc as plsc

_N = 10000
_E = 320000
_D = 128
_G = 128
_NB = 3
_NM = 3

_NC = 2
_NS = 16
_NW = _NC * _NS
_EPT = _E // _NW
_CH = 80
_NCH = _EPT // _CH
_PS = 2560
_NPASS = 4
_NP = _PS * _NPASS
_AR = _PS + 8
_RPS = _PS // _NS
_ZR = 40
_K = 2
_GE = _K * _CH
_TRASH = _EPT + _GE
_FL = _TRASH + 16


def _fill_rows(ref, nrows, value):
  def _st(i, _):
    ref[i // 8, pl.ds((i % 8) * 16, 16)] = jnp.full((16,), value, jnp.float32)
    return 0
  lax.fori_loop(0, nrows * 8, _st, 0)


def _zero_acc(acc_sh, zrow_v, s):
  def _z(k, _):
    pltpu.sync_copy(zrow_v, acc_sh.at[pl.ds(s * _RPS + k * _ZR, _ZR)])
    return 0
  lax.fori_loop(0, _RPS // _ZR, _z, 0)

  @pl.when(s == _NS - 1)
  def _last():
    pltpu.sync_copy(zrow_v.at[pl.ds(0, 8)], acc_sh.at[pl.ds(_PS, 8)])


def _filter_edges(dst_v, fdst_v, base, src_v, fsrc_v):
  lane = lax.iota(jnp.int32, 16)

  def _f(i, cnt):
    j = i // (_CH // 16)
    k = (i % (_CH // 16)) * 16
    d = dst_v[j, pl.ds(k, 16)]
    rel = d - base
    ok = jnp.logical_and(rel >= 0, rel < _PS)
    pref = plsc.cumsum(ok.astype(jnp.int32))
    pos = jnp.where(ok, cnt + pref - 1, _TRASH + lane)
    plsc.store_scatter(fdst_v, [pos], rel)
    sv = src_v[j, pl.ds(k, 16)]
    plsc.store_scatter(fsrc_v, [pos], sv)
    return cnt + jnp.max(pref)

  cnt = lax.fori_loop(0, _EPT // 16, _f, 0)
  spread = lax.iota(jnp.int32, 16) & 7
  for k in range(_GE // 16):
    fdst_v[pl.ds(cnt + k * 16, 16)] = spread + _PS
    fsrc_v[pl.ds(cnt + k * 16, 16)] = spread
  return jnp.maximum((cnt + _GE - 1) // _GE, 1)


def _readout(acc_sh, out_hbm, c, s, base):
  pltpu.sync_copy(acc_sh.at[pl.ds(s * _RPS, _RPS)],
                  out_hbm.at[c, pl.ds(base + s * _RPS, _RPS)])


def _make_segsum(build):
  mesh = plsc.VectorSubcoreMesh(core_axis_name="c", subcore_axis_name="s")
  out_type = [jax.ShapeDtypeStruct((_NC, _NP, _D), jnp.float32)]
  scratch = [
      pltpu.VMEM((_FL,), jnp.int32),
      pltpu.VMEM((_FL,), jnp.int32),
      pltpu.VMEM((_K, _CH, _D), jnp.float32),
      pltpu.VMEM((_K, _CH, _D), jnp.float32),
      pltpu.VMEM((_ZR, _D), jnp.float32),
      pltpu.VMEM_SHARED((_AR, _D), jnp.float32),
      pltpu.SemaphoreType.DMA,
      pltpu.SemaphoreType.DMA,
  ]
  scratch += [
      pltpu.VMEM((_NCH, _CH), jnp.int32),
      pltpu.VMEM((_NCH, _CH), jnp.int32),
  ]
  if build:
    out_type.append(jax.ShapeDtypeStruct((_NC, _NP, _D), jnp.float32))
    scratch.append(pltpu.VMEM((_CH, _D), jnp.float32))

  def body(x_hbm, src_hbm, dst_hbm, *rest):
    if build:
      (acc_out, deg_out,
       fsrc_v, fdst_v, buf_a, buf_b, zrow_v, acc_sh, gsem, ssem,
       src_v, dst_v, ones_v) = rest
    else:
      (acc_out,
       fsrc_v, fdst_v, buf_a, buf_b, zrow_v, acc_sh, gsem, ssem,
       src_v, dst_v) = rest
    c = lax.axis_index("c")
    s = lax.axis_index("s")
    wid = s * _NC + c

    _fill_rows(zrow_v, _ZR, 0.0)
    if build:
      _fill_rows(ones_v, _CH, 1.0)
    pltpu.sync_copy(src_hbm.at[wid], src_v)
    pltpu.sync_copy(dst_hbm.at[wid], dst_v)

    def _fire_gathers(j0, buf):
      for i in range(_K):
        pltpu.async_copy(
            x_hbm.at[fsrc_v.at[pl.ds(j0 * _GE + i * _CH, _CH)]],
            buf.at[i], gsem)

    def _drain_gathers(j0, buf):
      for i in range(_K):
        pltpu.make_async_copy(
            x_hbm.at[fsrc_v.at[pl.ds(j0 * _GE + i * _CH, _CH)]],
            buf.at[i], gsem).wait()

    def _pass(p, _):
      base = p * _PS
      _zero_acc(acc_sh, zrow_v, s)
      ngr = _filter_edges(dst_v, fdst_v, base, src_v, fsrc_v)
      plsc.subcore_barrier()

      _fire_gathers(0, buf_a)

      def _group(g, _):
        def _phase(buf_x, buf_y):
          _drain_gathers(g, buf_x)
          hs = [
              pltpu.async_copy(
                  buf_x.at[i],
                  acc_sh.at[fdst_v.at[pl.ds(g * _GE + i * _CH, _CH)]],
                  ssem, add=True)
              for i in range(_K)
          ]

          @pl.when(g + 1 < ngr)
          def _next():
            _fire_gathers(g + 1, buf_y)

          for h in hs:
            h.wait()

        @pl.when(g % 2 == 0)
        def _even():
          _phase(buf_a, buf_b)

        @pl.when(g % 2 == 1)
        def _odd():
          _phase(buf_b, buf_a)
        return 0
      lax.fori_loop(0, ngr, _group, 0)

      plsc.subcore_barrier()
      _readout(acc_sh, acc_out, c, s, base)

      if build:
        _zero_acc(acc_sh, zrow_v, s)
        plsc.subcore_barrier()

        def _dgroup(g, _):
          hs = [
              pltpu.async_copy(
                  ones_v,
                  acc_sh.at[fdst_v.at[pl.ds(g * _GE + i * _CH, _CH)]],
                  ssem, add=True)
              for i in range(_K)
          ]
          for h in hs:
            h.wait()
          return 0
        lax.fori_loop(0, ngr, _dgroup, 0)

        plsc.subcore_barrier()
        _readout(acc_sh, deg_out, c, s, base)
      return 0

    lax.fori_loop(0, _NPASS, _pass, 0)

  if not build:
    out_type = out_type[0]
  return pl.kernel(
      body,
      out_type=out_type,
      mesh=mesh,
      scratch_types=scratch,
      compiler_params=pltpu.CompilerParams(needs_layout_passes=False),
  )


_SEGSUM_DEG = _make_segsum(True)
_SEGSUM = _make_segsum(False)

_R = 2000


def _ln(t, g, b):
  m = jnp.mean(t, axis=-1, keepdims=True)
  xm = t - m
  v = jnp.mean(xm * xm, axis=-1, keepdims=True)
  return xm / jnp.sqrt(v + 1e-5) * g + b


def _elu(v):
  return jnp.where(v > 0, v, jnp.exp(v) - 1.0)


def _dense_body(x_ref, pa_ref, pb_ref, da_ref, db_ref,
                wl_ref, bl_ref, wr_ref, br_ref, g1_ref, be1_ref,
                w1_ref, b1_ref, w2_ref, b2_ref, g2_ref, be2_ref, o_ref):
  x = x_ref[...]
  sagg = pa_ref[0] + pb_ref[0]
  deg = jnp.maximum(da_ref[0][:, :1] + db_ref[0][:, :1], 1.0)
  mean = sagg / deg
  y = (jnp.dot(mean, wl_ref[...], preferred_element_type=jnp.float32)
       + bl_ref[...]
       + jnp.dot(x, wr_ref[...], preferred_element_type=jnp.float32)
       + br_ref[...])
  y1 = _ln(y + x, g1_ref[...], be1_ref[...])
  h = _elu(jnp.dot(y1, w1_ref[...], preferred_element_type=jnp.float32)
           + b1_ref[...])
  h = jnp.dot(h, w2_ref[...], preferred_element_type=jnp.float32) + b2_ref[...]
  o_ref[...] = _ln(h + y1, g2_ref[...], be2_ref[...])


def _dense_block(x, acc, degp, weights):
  row = lambda i: (i, 0)
  fixed = lambda i: (0, 0)
  in_specs = [
      pl.BlockSpec((_R, _D), row),
      pl.BlockSpec((1, _R, _D), lambda i: (0, i, 0)),
      pl.BlockSpec((1, _R, _D), lambda i: (1, i, 0)),
      pl.BlockSpec((1, _R, _D), lambda i: (0, i, 0)),
      pl.BlockSpec((1, _R, _D), lambda i: (1, i, 0)),
  ]
  for a in weights:
    in_specs.append(pl.BlockSpec(a.shape, fixed))
  return pl.pallas_call(
      _dense_body,
      grid=(_N // _R,),
      in_specs=in_specs,
      out_specs=pl.BlockSpec((_R, _D), row),
      out_shape=jax.ShapeDtypeStruct((_N, _D), jnp.float32),
  )(x, acc, acc, degp, degp, *weights)


def _dense_pool_body(x_ref, pa_ref, pb_ref, da_ref, db_ref, b_ref,
                     wl_ref, bl_ref, wr_ref, br_ref, g1_ref, be1_ref,
                     w1_ref, b1_ref, w2_ref, b2_ref, g2_ref, be2_ref,
                     m1_ref, c1_ref, m2_ref, c2_ref, m3_ref, c3_ref,
                     wh_ref, ch_ref, o_ref, acc_ref, cnt_ref):
  i = pl.program_id(0)

  @pl.when(i == 0)
  def _init():
    acc_ref[...] = jnp.zeros_like(acc_ref)
    cnt_ref[...] = jnp.zeros_like(cnt_ref)

  x = x_ref[...]
  sagg = pa_ref[0] + pb_ref[0]
  deg = jnp.maximum(da_ref[0][:, :1] + db_ref[0][:, :1], 1.0)
  mean = sagg / deg
  y = (jnp.dot(mean, wl_ref[...], preferred_element_type=jnp.float32)
       + bl_ref[...]
       + jnp.dot(x, wr_ref[...], preferred_element_type=jnp.float32)
       + br_ref[...])
  y1 = _ln(y + x, g1_ref[...], be1_ref[...])
  h = _elu(jnp.dot(y1, w1_ref[...], preferred_element_type=jnp.float32)
           + b1_ref[...])
  h = jnp.dot(h, w2_ref[...], preferred_element_type=jnp.float32) + b2_ref[...]
  res = _ln(h + y1, g2_ref[...], be2_ref[...])

  b = b_ref[...]
  gids = lax.broadcasted_iota(jnp.int32, (1, _G), 1).astype(jnp.float32)
  onehot = (b == gids).astype(jnp.float32)
  dn = (((0,), (0,)), ((), ()))
  acc_ref[...] += lax.dot_general(onehot, res, dn,
                                  preferred_element_type=jnp.float32)
  cnt_ref[...] += lax.dot_general(onehot, jnp.ones((_R, _D), jnp.float32), dn,
                                  preferred_element_type=jnp.float32)

  @pl.when(i == pl.num_programs(0) - 1)
  def _fin():
    pooled = acc_ref[...] / jnp.maximum(cnt_ref[...], 1.0)
    for w_r, c_r in ((m1_ref, c1_ref), (m2_ref, c2_ref), (m3_ref, c3_ref)):
      pooled = _elu(jnp.dot(pooled, w_r[...],
                            preferred_element_type=jnp.float32) + c_r[...])
    o_ref[...] = (jnp.dot(pooled, wh_ref[...],
                          preferred_element_type=jnp.float32) + ch_ref[...])


def _dense_pool_block(x, acc, degp, bf, weights):
  fixed = lambda i: (0, 0)
  split = pl.BlockSpec((1, _R, _D), lambda i: (0, i, 0))
  splitb = pl.BlockSpec((1, _R, _D), lambda i: (1, i, 0))
  in_specs = [
      pl.BlockSpec((_R, _D), lambda i: (i, 0)),
      split, splitb, split, splitb,
      pl.BlockSpec((_R, 1), lambda i: (i, 0)),
  ]
  for a in weights:
    in_specs.append(pl.BlockSpec(a.shape, fixed))
  return pl.pallas_call(
      _dense_pool_body,
      grid=(_N // _R,),
      in_specs=in_specs,
      out_specs=pl.BlockSpec((_G, _D), fixed),
      out_shape=jax.ShapeDtypeStruct((_G, _D), jnp.float32),
      scratch_shapes=[
          pltpu.VMEM((_G, _D), jnp.float32),
          pltpu.VMEM((_G, _D), jnp.float32),
      ],
  )(x, acc, acc, degp, degp, bf, *weights)


def kernel(x, edge_index, batch, params):
  src = edge_index[0].reshape(_NW, _NCH, _CH)
  dst = edge_index[1].reshape(_NW, _NCH, _CH)
  bf = batch.astype(jnp.float32).reshape(_N, 1)
  it = iter(params)

  blocks = []
  for _ in range(_NB):
    (wl, bl, wr, br, g1, be1, w1, b1, w2, b2, g2, be2) = (
        next(it) for _ in range(12))
    blocks.append((wl, bl.reshape(1, _D), wr, br.reshape(1, _D),
                   g1.reshape(1, _D), be1.reshape(1, _D),
                   w1, b1.reshape(1, _D), w2, b2.reshape(1, _D),
                   g2.reshape(1, _D), be2.reshape(1, _D)))
  mlp = []
  for _ in range(_NM):
    w = next(it)
    b = next(it)
    mlp += [w, b.reshape(1, _D)]
  wh = next(it)
  bh = next(it)
  whp = jnp.pad(wh, ((0, 0), (0, _D - 2)))
  bhp = jnp.pad(bh, (0, _D - 2)).reshape(1, _D)

  xcur = x
  degp = None
  for blk in range(_NB - 1):
    if degp is None:
      acc, degp = _SEGSUM_DEG(xcur, src, dst)
    else:
      acc = _SEGSUM(xcur, src, dst)
    xcur = _dense_block(xcur, acc, degp, blocks[blk])

  acc = _SEGSUM(xcur, src, dst)
  raw = _dense_pool_block(xcur, acc, degp, bf,
                          (*blocks[-1], *mlp, whp, bhp))[:, :2]
  out = raw.reshape(-1, 1, 2)
  return out[..., 0], out[..., 1]
